# baseline (speedup 1.0000x reference)
import functools

import jax
import jax.numpy as jnp
from jax import lax
from jax.experimental import pallas as pl
from jax.experimental.pallas import tpu as pltpu
from jax.experimental.pallas import tpu_sc as plsc

B, N, K, D = 4, 4096, 4096, 3
KT = 256
NK = K // KT
H1, H2, OUT = 512, 256, 256
IN = K + K * D
RW = 16

NC, NS = 2, 16
NW = NC * NS
CH = (B * K) // NW


def _nn_body(basis_ref, pc_ref, dists_ref, idx_ref):
    b = pl.program_id(0)
    c = basis_ref[...]
    p = pc_ref[0]
    pc2 = p[0:1, :] * p[0:1, :] + p[1:2, :] * p[1:2, :] + p[2:3, :] * p[2:3, :]
    bs2 = c[:, 0:1] * c[:, 0:1] + c[:, 1:2] * c[:, 1:2] + c[:, 2:3] * c[:, 2:3]
    cross = jnp.dot(c.astype(jnp.bfloat16), p.astype(jnp.bfloat16),
                    preferred_element_type=jnp.float32)
    raw = (pc2 + bs2) - 2.0 * cross
    mn = jnp.min(raw, axis=1, keepdims=True)
    thresh = jnp.maximum(mn, 0.0)
    iota = lax.broadcasted_iota(jnp.int32, (KT, N), 1)
    idx = jnp.min(jnp.where(raw <= thresh, iota, N), axis=1).astype(jnp.int32)
    dists_ref[0, 0, :] = jnp.sqrt(jnp.maximum(mn[:, 0], 1e-12))
    idx_ref[0, 0, :] = idx + b * N


def _nearest_tc(pc_pad, basis_pad):
    grid = (B, NK)
    out_shapes = (
        jax.ShapeDtypeStruct((B * NK, 1, KT), jnp.float32),
        jax.ShapeDtypeStruct((B * NK, 1, KT), jnp.int32),
    )
    dists3, idx3 = pl.pallas_call(
        _nn_body,
        grid=grid,
        in_specs=[
            pl.BlockSpec((KT, 8), lambda b, i: (i, 0)),
            pl.BlockSpec((1, 8, N), lambda b, i: (b, 0, 0)),
        ],
        out_specs=(
            pl.BlockSpec((1, 1, KT), lambda b, i: (b * NK + i, 0, 0)),
            pl.BlockSpec((1, 1, KT), lambda b, i: (b * NK + i, 0, 0)),
        ),
        out_shape=out_shapes,
    )(basis_pad, pc_pad)
    return dists3.reshape(B, K), idx3.reshape(B * K)


def _sc_gather_build():
    mesh = plsc.VectorSubcoreMesh(core_axis_name="c", subcore_axis_name="s")

    @functools.partial(
        pl.kernel,
        mesh=mesh,
        out_type=jax.ShapeDtypeStruct((B * K, RW), jnp.float32),
        compiler_params=pltpu.CompilerParams(use_tc_tiling_on_sc=False),
        scratch_types=[
            pltpu.VMEM((CH,), jnp.int32),
            pltpu.VMEM((CH, RW), jnp.float32),
            pltpu.SemaphoreType.DMA,
        ],
    )
    def sc_gather(rows_hbm, idx_hbm, out_hbm, idx_v, rows_v, sem):
        wid = lax.axis_index("s") * NC + lax.axis_index("c")
        base = wid * CH
        pltpu.sync_copy(idx_hbm.at[pl.ds(base, CH)], idx_v)
        pltpu.async_copy(rows_hbm.at[idx_v], rows_v, sem).wait()
        pltpu.sync_copy(rows_v, out_hbm.at[pl.ds(base, CH)])

    return sc_gather


_sc_gather_cache = []


def _sc_gather(*args):
    if not _sc_gather_cache:
        _sc_gather_cache.append(_sc_gather_build())
    return _sc_gather_cache[0](*args)


def _lrelu_k(x):
    return jnp.where(x >= 0, x, 0.2 * x)


def _mlp_body(distsT_ref, nearT_ref, bcol_ref, w1_ref, b1_ref, w2_ref, b2_ref,
              w3_ref, b3_ref, outT_ref, featT_ref):
    bf16, f32 = jnp.bfloat16, jnp.float32
    featT_ref[:K, :] = distsT_ref[...]
    featT_ref[K:, :] = nearT_ref[...] - bcol_ref[...]
    fT = featT_ref[...]
    h = jnp.dot(w1_ref[...].astype(bf16), fT.astype(bf16),
                preferred_element_type=f32)
    h = _lrelu_k(h + b1_ref[...])
    h = jnp.dot(w2_ref[...].astype(bf16), h.astype(bf16),
                preferred_element_type=f32)
    h = _lrelu_k(h + b2_ref[...])
    h = jnp.dot(w3_ref[...].astype(bf16), h.astype(bf16),
                preferred_element_type=f32)
    outT_ref[...] = h + b3_ref[...]


def _mlp_tc(distsT, nearT, bcol, w1, b1c, w2, b2c, w3, b3c):
    return pl.pallas_call(
        _mlp_body,
        out_shape=(
            jax.ShapeDtypeStruct((OUT, 8), jnp.float32),
            jax.ShapeDtypeStruct((IN, 8), jnp.float32),
        ),
    )(distsT, nearT, bcol, w1, b1c, w2, b2c, w3, b3c)


def kernel(point_cloud, basis, W1, b1, W2, b2, W3, b3):
    f32 = jnp.float32
    pc_t = point_cloud.transpose(0, 2, 1)
    pc_pad = jnp.zeros((B, 8, N), f32).at[:, :3, :].set(pc_t)
    basis_pad = jnp.zeros((K, 8), f32).at[:, :3].set(basis)
    pc_rows = jnp.zeros((B * N, RW), f32).at[:, :3].set(
        point_cloud.reshape(B * N, 3))

    dists, idx = _nearest_tc(pc_pad, basis_pad)

    near_rows = _sc_gather(pc_rows, idx)

    distsT = jnp.zeros((K, 8), f32).at[:, :B].set(dists.T)
    nearT = jnp.zeros((K * D, 8), f32).at[:, :B].set(
        near_rows.reshape(B, K, RW)[:, :, :D].transpose(1, 2, 0).reshape(K * D, B))
    outT, featT = _mlp_tc(
        distsT, nearT, basis.reshape(K * D, 1),
        W1, b1.reshape(H1, 1),
        W2, b2.reshape(H2, 1),
        W3, b3.reshape(OUT, 1),
    )
    return outT.T[:B], featT.T[:B, None, :]


# speedup vs baseline: 1.0032x; 1.0032x over previous
import functools

import jax
import jax.numpy as jnp
from jax import lax
from jax.experimental import pallas as pl
from jax.experimental.pallas import tpu as pltpu
from jax.experimental.pallas import tpu_sc as plsc

B, N, K, D = 4, 4096, 4096, 3
KT = 256
NK = K // KT
H1, H2, OUT = 512, 256, 256
IN = K + K * D
RW = 16

NC, NS = 2, 16
NW = NC * NS
CH = (B * K) // NW


def _nn_body(basis_ref, pc_ref, dists_ref, idx_ref):
    b = pl.program_id(0)
    c = basis_ref[...]
    p = pc_ref[0]
    pc2 = p[0:1, :] * p[0:1, :] + p[1:2, :] * p[1:2, :] + p[2:3, :] * p[2:3, :]
    bs2 = c[:, 0:1] * c[:, 0:1] + c[:, 1:2] * c[:, 1:2] + c[:, 2:3] * c[:, 2:3]
    cross = jnp.dot(c.astype(jnp.bfloat16), p.astype(jnp.bfloat16),
                    preferred_element_type=jnp.float32)
    raw = (pc2 + bs2) - 2.0 * cross
    mn = jnp.min(raw, axis=1, keepdims=True)
    thresh = jnp.maximum(mn, 0.0)
    iota = lax.broadcasted_iota(jnp.int32, (KT, N), 1)
    idx = jnp.min(jnp.where(raw <= thresh, iota, N), axis=1).astype(jnp.int32)
    dists_ref[0, 0, :] = jnp.sqrt(jnp.maximum(mn[:, 0], 1e-12))
    idx_ref[0, 0, :] = idx + b * N


def _nearest_tc(pc_pad, basis_pad):
    grid = (B, NK)
    out_shapes = (
        jax.ShapeDtypeStruct((B * NK, 1, KT), jnp.float32),
        jax.ShapeDtypeStruct((B * NK, 1, KT), jnp.int32),
    )
    dists3, idx3 = pl.pallas_call(
        _nn_body,
        grid=grid,
        in_specs=[
            pl.BlockSpec((KT, 8), lambda b, i: (i, 0)),
            pl.BlockSpec((1, 8, N), lambda b, i: (b, 0, 0)),
        ],
        out_specs=(
            pl.BlockSpec((1, 1, KT), lambda b, i: (b * NK + i, 0, 0)),
            pl.BlockSpec((1, 1, KT), lambda b, i: (b * NK + i, 0, 0)),
        ),
        out_shape=out_shapes,
    )(basis_pad, pc_pad)
    return dists3.reshape(B, K), idx3.reshape(B * K)


def _sc_gather_build():
    mesh = plsc.VectorSubcoreMesh(core_axis_name="c", subcore_axis_name="s")

    @functools.partial(
        pl.kernel,
        mesh=mesh,
        out_type=jax.ShapeDtypeStruct((B * K, RW), jnp.float32),
        compiler_params=pltpu.CompilerParams(use_tc_tiling_on_sc=False),
        scratch_types=[
            pltpu.VMEM((CH,), jnp.int32),
            pltpu.VMEM((CH, RW), jnp.float32),
            pltpu.SemaphoreType.DMA,
        ],
    )
    def sc_gather(rows_hbm, idx_hbm, out_hbm, idx_v, rows_v, sem):
        wid = lax.axis_index("s") * NC + lax.axis_index("c")
        base = wid * CH
        pltpu.sync_copy(idx_hbm.at[pl.ds(base, CH)], idx_v)
        pltpu.async_copy(rows_hbm.at[idx_v], rows_v, sem).wait()
        pltpu.sync_copy(rows_v, out_hbm.at[pl.ds(base, CH)])

    return sc_gather


_sc_gather_cache = []


def _sc_gather(*args):
    if not _sc_gather_cache:
        _sc_gather_cache.append(_sc_gather_build())
    return _sc_gather_cache[0](*args)


def _lrelu_k(x):
    return jnp.where(x >= 0, x, 0.2 * x)


INC = 2048
NSTEP = IN // INC


def _mlp_body(distsT_ref, nearT_ref, bcol_ref, w1_ref, b1_ref, w2_ref, b2_ref,
              w3_ref, b3_ref, outT_ref, featT_ref, acc_ref):
    i = pl.program_id(0)
    bf16, f32 = jnp.bfloat16, jnp.float32

    @pl.when(i < K // INC)
    def _():
        featT_ref[...] = distsT_ref[...]

    @pl.when(i >= K // INC)
    def _():
        featT_ref[...] = nearT_ref[...] - bcol_ref[...]

    part = jnp.dot(w1_ref[...].astype(bf16), featT_ref[...].astype(bf16),
                   preferred_element_type=f32)

    @pl.when(i == 0)
    def _():
        acc_ref[...] = part

    @pl.when(i > 0)
    def _():
        acc_ref[...] += part

    @pl.when(i == NSTEP - 1)
    def _():
        h = _lrelu_k(acc_ref[...] + b1_ref[...])
        h = jnp.dot(w2_ref[...].astype(bf16), h.astype(bf16),
                    preferred_element_type=f32)
        h = _lrelu_k(h + b2_ref[...])
        h = jnp.dot(w3_ref[...].astype(bf16), h.astype(bf16),
                    preferred_element_type=f32)
        outT_ref[...] = h + b3_ref[...]


def _mlp_tc(distsT, nearT, bcol, w1, b1c, w2, b2c, w3, b3c):
    nd = K // INC
    return pl.pallas_call(
        _mlp_body,
        grid=(NSTEP,),
        in_specs=[
            pl.BlockSpec((INC, 8), lambda i: (jnp.minimum(i, nd - 1), 0)),
            pl.BlockSpec((INC, 8), lambda i: (jnp.maximum(i - nd, 0), 0)),
            pl.BlockSpec((INC, 1), lambda i: (jnp.maximum(i - nd, 0), 0)),
            pl.BlockSpec((H1, INC), lambda i: (0, i)),
            pl.BlockSpec((H1, 1), lambda i: (0, 0)),
            pl.BlockSpec((H2, H1), lambda i: (0, 0)),
            pl.BlockSpec((H2, 1), lambda i: (0, 0)),
            pl.BlockSpec((OUT, H2), lambda i: (0, 0)),
            pl.BlockSpec((OUT, 1), lambda i: (0, 0)),
        ],
        out_specs=(
            pl.BlockSpec((OUT, 8), lambda i: (0, 0)),
            pl.BlockSpec((INC, 8), lambda i: (i, 0)),
        ),
        out_shape=(
            jax.ShapeDtypeStruct((OUT, 8), jnp.float32),
            jax.ShapeDtypeStruct((IN, 8), jnp.float32),
        ),
        scratch_shapes=[pltpu.VMEM((H1, 8), jnp.float32)],
    )(distsT, nearT, bcol, w1, b1c, w2, b2c, w3, b3c)


def kernel(point_cloud, basis, W1, b1, W2, b2, W3, b3):
    f32 = jnp.float32
    pc_t = point_cloud.transpose(0, 2, 1)
    pc_pad = jnp.zeros((B, 8, N), f32).at[:, :3, :].set(pc_t)
    basis_pad = jnp.zeros((K, 8), f32).at[:, :3].set(basis)
    pc_rows = jnp.zeros((B * N, RW), f32).at[:, :3].set(
        point_cloud.reshape(B * N, 3))

    dists, idx = _nearest_tc(pc_pad, basis_pad)

    near_rows = _sc_gather(pc_rows, idx)

    distsT = jnp.zeros((K, 8), f32).at[:, :B].set(dists.T)
    nearT = jnp.zeros((K * D, 8), f32).at[:, :B].set(
        near_rows.reshape(B, K, RW)[:, :, :D].transpose(1, 2, 0).reshape(K * D, B))
    outT, featT = _mlp_tc(
        distsT, nearT, basis.reshape(K * D, 1),
        W1, b1.reshape(H1, 1),
        W2, b2.reshape(H2, 1),
        W3, b3.reshape(OUT, 1),
    )
    return outT.T[:B], featT.T[:B, None, :]


# speedup vs baseline: 1.0820x; 1.0786x over previous
import functools

import jax
import jax.numpy as jnp
from jax import lax
from jax.experimental import pallas as pl
from jax.experimental.pallas import tpu as pltpu
from jax.experimental.pallas import tpu_sc as plsc

B, N, K, D = 4, 4096, 4096, 3
KT = 256
NK = K // KT
H1, H2, OUT = 512, 256, 256
IN = K + K * D
RW = 16

NC, NS = 2, 16
NW = NC * NS
CH = (B * K) // NW


def _nn_body(basis_ref, pc_ref, dists_ref, idx_ref):
    b = pl.program_id(0)
    c = basis_ref[...]
    p = pc_ref[0]
    pc2 = p[0:1, :] * p[0:1, :] + p[1:2, :] * p[1:2, :] + p[2:3, :] * p[2:3, :]
    bs2 = c[:, 0:1] * c[:, 0:1] + c[:, 1:2] * c[:, 1:2] + c[:, 2:3] * c[:, 2:3]
    cross = jnp.dot(c.astype(jnp.bfloat16), p.astype(jnp.bfloat16),
                    preferred_element_type=jnp.float32)
    raw = (pc2 + bs2) - 2.0 * cross
    mn = jnp.min(raw, axis=1, keepdims=True)
    thresh = jnp.maximum(mn, 0.0)
    iota = lax.broadcasted_iota(jnp.int32, (1, N), 1)
    idx = jnp.min(jnp.where(raw <= thresh, iota, N), axis=1).astype(jnp.int32)
    dists_ref[0, 0, :] = jnp.sqrt(jnp.maximum(mn[:, 0], 1e-12))
    idx_ref[0, 0, :] = idx + b * N


def _nearest_tc(pc_pad, basis_pad):
    grid = (B, NK)
    out_shapes = (
        jax.ShapeDtypeStruct((B * NK, 1, KT), jnp.float32),
        jax.ShapeDtypeStruct((B * NK, 1, KT), jnp.int32),
    )
    dists3, idx3 = pl.pallas_call(
        _nn_body,
        grid=grid,
        in_specs=[
            pl.BlockSpec((KT, 8), lambda b, i: (i, 0)),
            pl.BlockSpec((1, 8, N), lambda b, i: (b, 0, 0)),
        ],
        out_specs=(
            pl.BlockSpec((1, 1, KT), lambda b, i: (b * NK + i, 0, 0)),
            pl.BlockSpec((1, 1, KT), lambda b, i: (b * NK + i, 0, 0)),
        ),
        out_shape=out_shapes,
    )(basis_pad, pc_pad)
    return dists3.reshape(B, K), idx3.reshape(B * K)


def _sc_gather_build():
    mesh = plsc.VectorSubcoreMesh(core_axis_name="c", subcore_axis_name="s")

    @functools.partial(
        pl.kernel,
        mesh=mesh,
        out_type=jax.ShapeDtypeStruct((B * K, RW), jnp.float32),
        compiler_params=pltpu.CompilerParams(use_tc_tiling_on_sc=False),
        scratch_types=[
            pltpu.VMEM((CH,), jnp.int32),
            pltpu.VMEM((CH, RW), jnp.float32),
            pltpu.SemaphoreType.DMA,
        ],
    )
    def sc_gather(rows_hbm, idx_hbm, out_hbm, idx_v, rows_v, sem):
        wid = lax.axis_index("s") * NC + lax.axis_index("c")
        base = wid * CH
        pltpu.sync_copy(idx_hbm.at[pl.ds(base, CH)], idx_v)
        pltpu.async_copy(rows_hbm.at[idx_v], rows_v, sem).wait()
        pltpu.sync_copy(rows_v, out_hbm.at[pl.ds(base, CH)])

    return sc_gather


_sc_gather_cache = []


def _sc_gather(*args):
    if not _sc_gather_cache:
        _sc_gather_cache.append(_sc_gather_build())
    return _sc_gather_cache[0](*args)


def _lrelu_k(x):
    return jnp.where(x >= 0, x, 0.2 * x)


def _mlp_body(dists_ref, near_ref, bflat_ref, w1_ref, b1_ref, w2_ref, b2_ref,
              w3_ref, b3_ref, out_ref, feat_ref):
    bf16, f32 = jnp.bfloat16, jnp.float32
    feat = jnp.concatenate(
        [dists_ref[...], near_ref[...] - bflat_ref[...]], axis=1)
    feat_ref[...] = feat
    h = jnp.dot(feat.astype(bf16), w1_ref[...], preferred_element_type=f32)
    h = _lrelu_k(h + b1_ref[...])
    h = jnp.dot(h.astype(bf16), w2_ref[...], preferred_element_type=f32)
    h = _lrelu_k(h + b2_ref[...])
    h = jnp.dot(h.astype(bf16), w3_ref[...], preferred_element_type=f32)
    out_ref[...] = h + b3_ref[...]


def _mlp_tc(dists_p, near_p, bflat, w1t, b1r, w2t, b2r, w3t, b3r):
    return pl.pallas_call(
        _mlp_body,
        out_shape=(
            jax.ShapeDtypeStruct((8, OUT), jnp.float32),
            jax.ShapeDtypeStruct((8, IN), jnp.float32),
        ),
    )(dists_p, near_p, bflat, w1t, b1r, w2t, b2r, w3t, b3r)


def kernel(point_cloud, basis, W1, b1, W2, b2, W3, b3):
    f32 = jnp.float32
    pc_t = point_cloud.transpose(0, 2, 1)
    pc_pad = jnp.zeros((B, 8, N), f32).at[:, :3, :].set(pc_t)
    basis_pad = jnp.zeros((K, 8), f32).at[:, :3].set(basis)
    pc_rows = jnp.zeros((B * N, RW), f32).at[:, :3].set(
        point_cloud.reshape(B * N, 3))

    dists, idx = _nearest_tc(pc_pad, basis_pad)

    near_rows = _sc_gather(pc_rows, idx)
    near3 = near_rows[:, :D].reshape(B, K * D)

    dists_p = jnp.zeros((8, K), f32).at[:B].set(dists)
    near_p = jnp.zeros((8, K * D), f32).at[:B].set(near3)
    out, feat = _mlp_tc(
        dists_p, near_p, basis.reshape(1, K * D),
        W1.T.astype(jnp.bfloat16), b1.reshape(1, H1),
        W2.T.astype(jnp.bfloat16), b2.reshape(1, H2),
        W3.T.astype(jnp.bfloat16), b3.reshape(1, OUT),
    )
    return out[:B], feat[:B, None, :]


# speedup vs baseline: 1.1134x; 1.0290x over previous
import functools

import jax
import jax.numpy as jnp
from jax import lax
from jax.experimental import pallas as pl
from jax.experimental.pallas import tpu as pltpu
from jax.experimental.pallas import tpu_sc as plsc

B, N, K, D = 4, 4096, 4096, 3
KT = 256
NK = K // KT
H1, H2, OUT = 512, 256, 256
IN = K + K * D
RW = 16

NC, NS = 2, 16
NW = NC * NS
CH = (B * K) // NW


def _nn_body(basis_ref, pc_ref, dists_ref, idx_ref):
    b = pl.program_id(0)
    c = basis_ref[...]
    p = pc_ref[0]
    pc2 = p[0:1, :] * p[0:1, :] + p[1:2, :] * p[1:2, :] + p[2:3, :] * p[2:3, :]
    bs2 = c[:, 0:1] * c[:, 0:1] + c[:, 1:2] * c[:, 1:2] + c[:, 2:3] * c[:, 2:3]
    crossn = jnp.dot((-2.0 * c).astype(jnp.bfloat16), p.astype(jnp.bfloat16),
                     preferred_element_type=jnp.float32)
    raw = (pc2 + bs2) + crossn
    mn = jnp.min(raw, axis=1, keepdims=True)
    thresh = jnp.maximum(mn, 0.0)
    fiota = lax.broadcasted_iota(jnp.int32, (1, N), 1).astype(jnp.float32)
    fidx = jnp.min(jnp.where(raw <= thresh, fiota, jnp.float32(N)), axis=1)
    dists_ref[0, 0, :] = jnp.sqrt(jnp.maximum(mn[:, 0], 1e-12))
    idx_ref[0, 0, :] = fidx.astype(jnp.int32) + b * N


def _nearest_tc(pc_pad, basis_pad):
    grid = (B, NK)
    out_shapes = (
        jax.ShapeDtypeStruct((B * NK, 1, KT), jnp.float32),
        jax.ShapeDtypeStruct((B * NK, 1, KT), jnp.int32),
    )
    dists3, idx3 = pl.pallas_call(
        _nn_body,
        grid=grid,
        in_specs=[
            pl.BlockSpec((KT, 8), lambda b, i: (i, 0)),
            pl.BlockSpec((1, 8, N), lambda b, i: (b, 0, 0)),
        ],
        out_specs=(
            pl.BlockSpec((1, 1, KT), lambda b, i: (b * NK + i, 0, 0)),
            pl.BlockSpec((1, 1, KT), lambda b, i: (b * NK + i, 0, 0)),
        ),
        out_shape=out_shapes,
    )(basis_pad, pc_pad)
    return dists3.reshape(B, K), idx3.reshape(B * K)


def _sc_gather_build():
    mesh = plsc.VectorSubcoreMesh(core_axis_name="c", subcore_axis_name="s")

    @functools.partial(
        pl.kernel,
        mesh=mesh,
        out_type=jax.ShapeDtypeStruct((B * K, RW), jnp.float32),
        compiler_params=pltpu.CompilerParams(use_tc_tiling_on_sc=False),
        scratch_types=[
            pltpu.VMEM((CH,), jnp.int32),
            pltpu.VMEM((CH, RW), jnp.float32),
            pltpu.SemaphoreType.DMA,
        ],
    )
    def sc_gather(rows_hbm, idx_hbm, out_hbm, idx_v, rows_v, sem):
        wid = lax.axis_index("s") * NC + lax.axis_index("c")
        base = wid * CH
        pltpu.sync_copy(idx_hbm.at[pl.ds(base, CH)], idx_v)
        pltpu.async_copy(rows_hbm.at[idx_v], rows_v, sem).wait()
        pltpu.sync_copy(rows_v, out_hbm.at[pl.ds(base, CH)])

    return sc_gather


_sc_gather_cache = []


def _sc_gather(*args):
    if not _sc_gather_cache:
        _sc_gather_cache.append(_sc_gather_build())
    return _sc_gather_cache[0](*args)


def _lrelu_k(x):
    return jnp.where(x >= 0, x, 0.2 * x)


def _mlp_body(dists_ref, near_ref, bflat_ref, w1_ref, b1_ref, w2_ref, b2_ref,
              w3_ref, b3_ref, out_ref, feat_ref):
    bf16, f32 = jnp.bfloat16, jnp.float32
    feat = jnp.concatenate(
        [dists_ref[...], near_ref[...] - bflat_ref[...]], axis=1)
    feat_ref[...] = feat
    h = jnp.dot(feat.astype(bf16), w1_ref[...], preferred_element_type=f32)
    h = _lrelu_k(h + b1_ref[...])
    h = jnp.dot(h.astype(bf16), w2_ref[...], preferred_element_type=f32)
    h = _lrelu_k(h + b2_ref[...])
    h = jnp.dot(h.astype(bf16), w3_ref[...], preferred_element_type=f32)
    out_ref[...] = h + b3_ref[...]


def _mlp_tc(dists_p, near_p, bflat, w1t, b1r, w2t, b2r, w3t, b3r):
    return pl.pallas_call(
        _mlp_body,
        out_shape=(
            jax.ShapeDtypeStruct((8, OUT), jnp.float32),
            jax.ShapeDtypeStruct((8, IN), jnp.float32),
        ),
    )(dists_p, near_p, bflat, w1t, b1r, w2t, b2r, w3t, b3r)


def kernel(point_cloud, basis, W1, b1, W2, b2, W3, b3):
    f32 = jnp.float32
    pc_t = point_cloud.transpose(0, 2, 1)
    pc_pad = jnp.zeros((B, 8, N), f32).at[:, :3, :].set(pc_t)
    basis_pad = jnp.zeros((K, 8), f32).at[:, :3].set(basis)
    pc_rows = jnp.zeros((B * N, RW), f32).at[:, :3].set(
        point_cloud.reshape(B * N, 3))

    dists, idx = _nearest_tc(pc_pad, basis_pad)

    near_rows = _sc_gather(pc_rows, idx)
    near3 = near_rows[:, :D].reshape(B, K * D)

    dists_p = jnp.zeros((8, K), f32).at[:B].set(dists)
    near_p = jnp.zeros((8, K * D), f32).at[:B].set(near3)
    out, feat = _mlp_tc(
        dists_p, near_p, basis.reshape(1, K * D),
        W1.T.astype(jnp.bfloat16), b1.reshape(1, H1),
        W2.T.astype(jnp.bfloat16), b2.reshape(1, H2),
        W3.T.astype(jnp.bfloat16), b3.reshape(1, OUT),
    )
    return out[:B], feat[:B, None, :]
